# baseline (speedup 1.0000x reference)
import functools

import jax
import jax.numpy as jnp
from jax.experimental import pallas as pl
from jax.experimental.pallas import tpu as pltpu

TOPK = 60
THRESHOLD = 0.15
NMS_TH = 0.05
NMS_TOPK = 20
REG = 36
NCH = 3 * REG
FD = 24
N = FD * FD * FD
STRIDE = 4.0
NEG = -jnp.inf
PAD = 64


def _detect_kernel(cls_ref, shape_ref, off_ref, out_ref):
    logits = cls_ref[0]
    s = jax.nn.sigmoid(logits)

    r_iota = jax.lax.broadcasted_iota(jnp.int32, (NCH, 128), 0)
    l_iota = jax.lax.broadcasted_iota(jnp.int32, (NCH, 128), 1)
    n3 = r_iota * 128 + l_iota

    lane64 = jax.lax.broadcasted_iota(jnp.int32, (1, PAD), 1)

    def topk_body(k, carry):
        S, idxs, scs = carry
        m = jnp.max(S)
        cand = jnp.where(S == m, n3, jnp.int32(1 << 30))
        i = jnp.min(cand)
        S = jnp.where(n3 == i, NEG, S)
        idxs = jnp.where(lane64 == k, i, idxs)
        scs = jnp.where(lane64 == k, m, scs)
        return S, idxs, scs

    idxs0 = jnp.zeros((1, PAD), jnp.int32)
    scs0 = jnp.full((1, PAD), NEG, jnp.float32)
    _, idxs, ts = jax.lax.fori_loop(0, TOPK, topk_body, (s, idxs0, scs0))

    g_iota = jax.lax.broadcasted_iota(jnp.int32, (N, PAD), 0)
    onehot = (g_iota == idxs).astype(jnp.float32)
    dist_sel = jax.lax.dot_general(
        shape_ref[0], onehot, (((1,), (0,)), ((), ())),
        precision=jax.lax.Precision.HIGHEST)
    off_sel = jax.lax.dot_general(
        off_ref[0], onehot, (((1,), (0,)), ((), ())),
        precision=jax.lax.Precision.HIGHEST)

    az = (idxs // (FD * FD)).astype(jnp.float32)
    ay = ((idxs // FD) % FD).astype(jnp.float32)
    ax = (idxs % FD).astype(jnp.float32)

    proj_iota = jax.lax.broadcasted_iota(
        jnp.int32, (REG, PAD), 0).astype(jnp.float32)

    def project_rows(c):
        blk = dist_sel[c * REG:(c + 1) * REG, :]
        m = jnp.max(blk, axis=0, keepdims=True)
        e = jnp.exp(blk - m)
        p = e / jnp.sum(e, axis=0, keepdims=True)
        return jnp.sum(p * proj_iota, axis=0, keepdims=True)

    shp_z = project_rows(0)
    shp_y = project_rows(1)
    shp_x = project_rows(2)

    cz = (az + off_sel[0:1, :]) * STRIDE
    cy = (ay + off_sel[1:2, :]) * STRIDE
    cx = (ax + off_sel[2:3, :]) * STRIDE
    sz = shp_z * STRIDE
    sy = shp_y * STRIDE
    sx = shp_x * STRIDE

    loz, hiz = cz - sz * 0.5, cz + sz * 0.5
    loy, hiy = cy - sy * 0.5, cy + sy * 0.5
    lox, hix = cx - sx * 0.5, cx + sx * 0.5
    vol = sz * sy * sx

    out_ref[0] = jnp.full((TOPK, 8), -1.0, jnp.float32)
    s_cur = jnp.where(ts > THRESHOLD, ts, NEG)
    col8 = jax.lax.broadcasted_iota(jnp.int32, (1, 1, 8), 2)

    for k in range(NMS_TOPK):
        m = jnp.max(s_cur)
        cand = jnp.where(s_cur == m, lane64, jnp.int32(1 << 30))
        i = jnp.min(cand)
        ok = m > NEG
        sel = lane64 == i

        def pick(v):
            return jnp.sum(jnp.where(sel, v, 0.0))

        bz, by, bx = pick(cz), pick(cy), pick(cx)
        bsz, bsy, bsx = pick(sz), pick(sy), pick(sx)
        blz, bhz = bz - bsz * 0.5, bz + bsz * 0.5
        bly, bhy = by - bsy * 0.5, by + bsy * 0.5
        blx, bhx = bx - bsx * 0.5, bx + bsx * 0.5
        iz = jnp.clip(jnp.minimum(bhz, hiz) - jnp.maximum(blz, loz), 0.0, None)
        iy = jnp.clip(jnp.minimum(bhy, hiy) - jnp.maximum(bly, loy), 0.0, None)
        ix = jnp.clip(jnp.minimum(bhx, hix) - jnp.maximum(blx, lox), 0.0, None)
        inter = iz * iy * ix
        iou = inter / (bsz * bsy * bsx + vol - inter + 1e-8)
        supp = (iou > NMS_TH) | sel
        s_cur = jnp.where(ok & supp, NEG, s_cur)

        vals = jnp.where(col8 == 0, 1.0,
               jnp.where(col8 == 1, m,
               jnp.where(col8 == 2, bz,
               jnp.where(col8 == 3, by,
               jnp.where(col8 == 4, bx,
               jnp.where(col8 == 5, bsz,
               jnp.where(col8 == 6, bsy, bsx)))))))
        row = jnp.where(ok, vals, -1.0)
        out_ref[0:1, k:k + 1, :] = row


@jax.jit
def kernel(Cls, Shape, Offset):
    B = Cls.shape[0]
    cls_r = Cls.reshape(B, NCH, 128)
    shape_r = Shape.reshape(B, NCH, N)
    off_r = Offset.reshape(B, 3, N)
    return pl.pallas_call(
        _detect_kernel,
        grid=(B,),
        in_specs=[
            pl.BlockSpec((1, NCH, 128), lambda j: (j, 0, 0)),
            pl.BlockSpec((1, NCH, N), lambda j: (j, 0, 0)),
            pl.BlockSpec((1, 3, N), lambda j: (j, 0, 0)),
        ],
        out_specs=pl.BlockSpec((1, TOPK, 8), lambda j: (j, 0, 0)),
        out_shape=jax.ShapeDtypeStruct((B, TOPK, 8), jnp.float32),
        compiler_params=pltpu.CompilerParams(
            dimension_semantics=("parallel",)),
    )(cls_r, shape_r, off_r)


# speedup vs baseline: 2.3456x; 2.3456x over previous
import jax
import jax.numpy as jnp
from jax.experimental import pallas as pl
from jax.experimental.pallas import tpu as pltpu

TOPK = 60
THRESHOLD = 0.15
NMS_TH = 0.05
NMS_TOPK = 20
REG = 36
NCH = 3 * REG
FD = 24
N = FD * FD * FD
STRIDE = 4.0
NEG = -jnp.inf
BIG = 1 << 30
PAD = 64


def _topk_kernel(cls_ref, ts_ref, idx_ref):
    B = cls_ref.shape[0]
    S = jax.nn.sigmoid(cls_ref[...])
    r3 = jax.lax.broadcasted_iota(jnp.int32, (B, NCH, 128), 1)
    l3 = jax.lax.broadcasted_iota(jnp.int32, (B, NCH, 128), 2)
    n3 = r3 * 128 + l3
    lane = jax.lax.broadcasted_iota(jnp.int32, (B, PAD), 1)

    def body(k, carry):
        S, tsv, idxv = carry
        m = jnp.max(jnp.max(S, axis=1), axis=1, keepdims=True)
        cand = jnp.where(S == m[:, None, :], n3, BIG)
        i = jnp.min(jnp.min(cand, axis=1), axis=1, keepdims=True)
        S = jnp.where(n3 == i[:, None, :], NEG, S)
        tsv = jnp.where(lane == k, m, tsv)
        idxv = jnp.where(lane == k, i, idxv)
        return S, tsv, idxv

    tsv0 = jnp.full((B, PAD), NEG, jnp.float32)
    idxv0 = jnp.zeros((B, PAD), jnp.int32)
    _, tsv, idxv = jax.lax.fori_loop(0, TOPK, body, (S, tsv0, idxv0))
    ts_ref[...] = tsv
    idx_ref[...] = idxv


def _gather_kernel(idxv_ref, shape_ref, off_ref, boxes_ref):
    idxs = idxv_ref[0]
    g_iota = jax.lax.broadcasted_iota(jnp.int32, (N, PAD), 0)
    onehot = (g_iota == idxs).astype(jnp.float32)
    dsel = jax.lax.dot_general(
        shape_ref[0], onehot, (((1,), (0,)), ((), ())),
        precision=jax.lax.Precision.HIGHEST)

    proj_iota = jax.lax.broadcasted_iota(
        jnp.int32, (REG, PAD), 0).astype(jnp.float32)

    def project_rows(c):
        blk = dsel[c * REG:(c + 1) * REG, :]
        m = jnp.max(blk, axis=0, keepdims=True)
        e = jnp.exp(blk - m)
        p = e / jnp.sum(e, axis=0, keepdims=True)
        return jnp.sum(p * proj_iota, axis=0, keepdims=True)

    az = (idxs // (FD * FD)).astype(jnp.float32)
    ay = ((idxs // FD) % FD).astype(jnp.float32)
    ax = (idxs % FD).astype(jnp.float32)

    off_sel = jax.lax.dot_general(
        off_ref[0], onehot, (((1,), (0,)), ((), ())),
        precision=jax.lax.Precision.HIGHEST)

    cz = (az + off_sel[0:1, :]) * STRIDE
    cy = (ay + off_sel[1:2, :]) * STRIDE
    cx = (ax + off_sel[2:3, :]) * STRIDE
    sz = project_rows(0) * STRIDE
    sy = project_rows(1) * STRIDE
    sx = project_rows(2) * STRIDE

    boxes_ref[0, 0:1, :] = cz
    boxes_ref[0, 1:2, :] = cy
    boxes_ref[0, 2:3, :] = cx
    boxes_ref[0, 3:4, :] = sz
    boxes_ref[0, 4:5, :] = sy
    boxes_ref[0, 5:6, :] = sx
    boxes_ref[0, 6:8, :] = jnp.zeros((2, PAD), jnp.float32)


def _nms_kernel(ts_ref, boxes_ref, out_ref):
    B = ts_ref.shape[0]
    ts = ts_ref[...]
    cz = boxes_ref[:, 0, :]
    cy = boxes_ref[:, 1, :]
    cx = boxes_ref[:, 2, :]
    sz = boxes_ref[:, 3, :]
    sy = boxes_ref[:, 4, :]
    sx = boxes_ref[:, 5, :]
    loz, hiz = cz - sz * 0.5, cz + sz * 0.5
    loy, hiy = cy - sy * 0.5, cy + sy * 0.5
    lox, hix = cx - sx * 0.5, cx + sx * 0.5
    vol = sz * sy * sx

    lane = jax.lax.broadcasted_iota(jnp.int32, (B, PAD), 1)
    col8 = jax.lax.broadcasted_iota(jnp.int32, (1, 1, 8), 2)
    out_ref[...] = jnp.full((B, TOPK, 8), -1.0, jnp.float32)

    s_cur = jnp.where(ts > THRESHOLD, ts, NEG)
    for k in range(NMS_TOPK):
        m = jnp.max(s_cur, axis=1, keepdims=True)
        i = jnp.min(jnp.where(s_cur == m, lane, BIG), axis=1, keepdims=True)
        ok = m > NEG
        sel = lane == i

        def pick(v):
            return jnp.sum(jnp.where(sel, v, 0.0), axis=1, keepdims=True)

        bz, by, bx = pick(cz), pick(cy), pick(cx)
        bsz, bsy, bsx = pick(sz), pick(sy), pick(sx)
        iz = jnp.clip(jnp.minimum(bz + bsz * 0.5, hiz)
                      - jnp.maximum(bz - bsz * 0.5, loz), 0.0, None)
        iy = jnp.clip(jnp.minimum(by + bsy * 0.5, hiy)
                      - jnp.maximum(by - bsy * 0.5, loy), 0.0, None)
        ix = jnp.clip(jnp.minimum(bx + bsx * 0.5, hix)
                      - jnp.maximum(bx - bsx * 0.5, lox), 0.0, None)
        inter = iz * iy * ix
        iou = inter / (bsz * bsy * bsx + vol - inter + 1e-8)
        supp = (iou > NMS_TH) | sel
        s_cur = jnp.where(ok & supp, NEG, s_cur)

        def col(c, v):
            return jnp.where(col8 == c, v[:, :, None], 0.0)

        vals = (col(0, jnp.ones_like(m)) + col(1, m) + col(2, bz)
                + col(3, by) + col(4, bx) + col(5, bsz) + col(6, bsy)
                + col(7, bsx))
        out_ref[:, k:k + 1, :] = jnp.where(ok[:, :, None], vals, -1.0)


@jax.jit
def kernel(Cls, Shape, Offset):
    B = Cls.shape[0]
    cls_r = Cls.reshape(B, NCH, 128)
    shape_r = Shape.reshape(B, NCH, N)
    off_r = Offset.reshape(B, 3, N)

    ts, idx = pl.pallas_call(
        _topk_kernel,
        out_shape=(jax.ShapeDtypeStruct((B, PAD), jnp.float32),
                   jax.ShapeDtypeStruct((B, PAD), jnp.int32)),
    )(cls_r)

    boxes = pl.pallas_call(
        _gather_kernel,
        grid=(B,),
        in_specs=[
            pl.BlockSpec((1, 1, PAD), lambda j: (j, 0, 0)),
            pl.BlockSpec((1, NCH, N), lambda j: (j, 0, 0)),
            pl.BlockSpec((1, 3, N), lambda j: (j, 0, 0)),
        ],
        out_specs=pl.BlockSpec((1, 8, PAD), lambda j: (j, 0, 0)),
        out_shape=jax.ShapeDtypeStruct((B, 8, PAD), jnp.float32),
        compiler_params=pltpu.CompilerParams(
            dimension_semantics=("parallel",)),
    )(idx.reshape(B, 1, PAD), shape_r, off_r)

    return pl.pallas_call(
        _nms_kernel,
        out_shape=jax.ShapeDtypeStruct((B, TOPK, 8), jnp.float32),
    )(ts, boxes)


# speedup vs baseline: 2.5660x; 1.0940x over previous
import jax
import jax.numpy as jnp
from jax.experimental import pallas as pl
from jax.experimental.pallas import tpu as pltpu

TOPK = 60
THRESHOLD = 0.15
NMS_TH = 0.05
NMS_TOPK = 20
REG = 36
NCH = 3 * REG
FD = 24
N = FD * FD * FD
STRIDE = 4.0
NEG = -jnp.inf
BIG = 1 << 30
PAD = 64


def _topk_kernel(cls_ref, ts_ref, idx_ref):
    B = cls_ref.shape[0]
    S = jax.nn.sigmoid(cls_ref[...])
    r3 = jax.lax.broadcasted_iota(jnp.int32, (B, NCH, 128), 1)
    l3 = jax.lax.broadcasted_iota(jnp.int32, (B, NCH, 128), 2)
    n3 = r3 * 128 + l3
    lane = jax.lax.broadcasted_iota(jnp.int32, (B, PAD), 1)

    def body(k, carry):
        S, tsv, idxv = carry
        m = jnp.max(jnp.max(S, axis=1), axis=1, keepdims=True)
        cand = jnp.where(S == m[:, None, :], n3, BIG)
        i = jnp.min(jnp.min(cand, axis=1), axis=1, keepdims=True)
        S = jnp.where(n3 == i[:, None, :], NEG, S)
        tsv = jnp.where(lane == k, m, tsv)
        idxv = jnp.where(lane == k, i, idxv)
        return S, tsv, idxv

    tsv0 = jnp.full((B, PAD), NEG, jnp.float32)
    idxv0 = jnp.zeros((B, PAD), jnp.int32)
    _, tsv, idxv = jax.lax.fori_loop(0, TOPK, body, (S, tsv0, idxv0))
    ts_ref[...] = tsv
    idx_ref[...] = idxv


def _gather_kernel(idx_sref, idxv_ref, shape_hbm, off_hbm, boxes_ref,
                   gsh, gof, sem):
    jj = pl.program_id(0)
    copies = []
    for k in range(TOPK):
        ck = idx_sref[jj, k] // FD
        cp = pltpu.make_async_copy(
            shape_hbm.at[jj, :, ck, :], gsh.at[:, k, :], sem)
        cp.start()
        copies.append(cp)
        cpo = pltpu.make_async_copy(
            off_hbm.at[jj, :, ck, :], gof.at[:, k, :], sem)
        cpo.start()
        copies.append(cpo)
    gsh[:, TOPK:, :] = jnp.zeros((NCH, PAD - TOPK, FD), jnp.float32)
    gof[:, TOPK:, :] = jnp.zeros((3, PAD - TOPK, FD), jnp.float32)
    for cp in copies:
        cp.wait()

    idxs = idxv_ref[0]
    xsel = (jax.lax.broadcasted_iota(jnp.int32, (1, PAD, FD), 2)
            == (idxs % FD)[:, :, None]).astype(jnp.float32)
    dsel = jnp.sum(gsh[...] * xsel, axis=2)
    off_sel = jnp.sum(gof[...] * xsel, axis=2)

    proj_iota = jax.lax.broadcasted_iota(
        jnp.int32, (REG, PAD), 0).astype(jnp.float32)

    def project_rows(c):
        blk = dsel[c * REG:(c + 1) * REG, :]
        m = jnp.max(blk, axis=0, keepdims=True)
        e = jnp.exp(blk - m)
        p = e / jnp.sum(e, axis=0, keepdims=True)
        return jnp.sum(p * proj_iota, axis=0, keepdims=True)

    az = (idxs // (FD * FD)).astype(jnp.float32)
    ay = ((idxs // FD) % FD).astype(jnp.float32)
    ax = (idxs % FD).astype(jnp.float32)

    cz = (az + off_sel[0:1, :]) * STRIDE
    cy = (ay + off_sel[1:2, :]) * STRIDE
    cx = (ax + off_sel[2:3, :]) * STRIDE
    sz = project_rows(0) * STRIDE
    sy = project_rows(1) * STRIDE
    sx = project_rows(2) * STRIDE

    boxes_ref[0, 0:1, :] = cz
    boxes_ref[0, 1:2, :] = cy
    boxes_ref[0, 2:3, :] = cx
    boxes_ref[0, 3:4, :] = sz
    boxes_ref[0, 4:5, :] = sy
    boxes_ref[0, 5:6, :] = sx
    boxes_ref[0, 6:8, :] = jnp.zeros((2, PAD), jnp.float32)


def _nms_kernel(ts_ref, boxes_ref, out_ref):
    B = ts_ref.shape[0]
    ts = ts_ref[...]
    cz = boxes_ref[:, 0, :]
    cy = boxes_ref[:, 1, :]
    cx = boxes_ref[:, 2, :]
    sz = boxes_ref[:, 3, :]
    sy = boxes_ref[:, 4, :]
    sx = boxes_ref[:, 5, :]
    loz, hiz = cz - sz * 0.5, cz + sz * 0.5
    loy, hiy = cy - sy * 0.5, cy + sy * 0.5
    lox, hix = cx - sx * 0.5, cx + sx * 0.5
    vol = sz * sy * sx

    lane = jax.lax.broadcasted_iota(jnp.int32, (B, PAD), 1)
    col8 = jax.lax.broadcasted_iota(jnp.int32, (1, 1, 8), 2)
    out_ref[...] = jnp.full((B, TOPK, 8), -1.0, jnp.float32)

    s_cur = jnp.where(ts > THRESHOLD, ts, NEG)
    for k in range(NMS_TOPK):
        m = jnp.max(s_cur, axis=1, keepdims=True)
        i = jnp.min(jnp.where(s_cur == m, lane, BIG), axis=1, keepdims=True)
        ok = m > NEG
        sel = lane == i

        def pick(v):
            return jnp.sum(jnp.where(sel, v, 0.0), axis=1, keepdims=True)

        bz, by, bx = pick(cz), pick(cy), pick(cx)
        bsz, bsy, bsx = pick(sz), pick(sy), pick(sx)
        iz = jnp.clip(jnp.minimum(bz + bsz * 0.5, hiz)
                      - jnp.maximum(bz - bsz * 0.5, loz), 0.0, None)
        iy = jnp.clip(jnp.minimum(by + bsy * 0.5, hiy)
                      - jnp.maximum(by - bsy * 0.5, loy), 0.0, None)
        ix = jnp.clip(jnp.minimum(bx + bsx * 0.5, hix)
                      - jnp.maximum(bx - bsx * 0.5, lox), 0.0, None)
        inter = iz * iy * ix
        iou = inter / (bsz * bsy * bsx + vol - inter + 1e-8)
        supp = (iou > NMS_TH) | sel
        s_cur = jnp.where(ok & supp, NEG, s_cur)

        def col(c, v):
            return jnp.where(col8 == c, v[:, :, None], 0.0)

        vals = (col(0, jnp.ones_like(m)) + col(1, m) + col(2, bz)
                + col(3, by) + col(4, bx) + col(5, bsz) + col(6, bsy)
                + col(7, bsx))
        out_ref[:, k:k + 1, :] = jnp.where(ok[:, :, None], vals, -1.0)


@jax.jit
def kernel(Cls, Shape, Offset):
    B = Cls.shape[0]
    cls_r = Cls.reshape(B, NCH, 128)

    ts, idx = pl.pallas_call(
        _topk_kernel,
        out_shape=(jax.ShapeDtypeStruct((B, PAD), jnp.float32),
                   jax.ShapeDtypeStruct((B, PAD), jnp.int32)),
    )(cls_r)

    boxes = pl.pallas_call(
        _gather_kernel,
        grid_spec=pltpu.PrefetchScalarGridSpec(
            num_scalar_prefetch=1,
            grid=(B,),
            in_specs=[
                pl.BlockSpec((1, 1, PAD), lambda j, iref: (j, 0, 0)),
                pl.BlockSpec(memory_space=pltpu.MemorySpace.HBM),
                pl.BlockSpec(memory_space=pltpu.MemorySpace.HBM),
            ],
            out_specs=pl.BlockSpec((1, 8, PAD), lambda j, iref: (j, 0, 0)),
            scratch_shapes=[
                pltpu.VMEM((NCH, PAD, FD), jnp.float32),
                pltpu.VMEM((3, PAD, FD), jnp.float32),
                pltpu.SemaphoreType.DMA,
            ],
        ),
        out_shape=jax.ShapeDtypeStruct((B, 8, PAD), jnp.float32),
        compiler_params=pltpu.CompilerParams(
            dimension_semantics=("arbitrary",)),
    )(idx, idx.reshape(B, 1, PAD),
      Shape.reshape(B, NCH, FD * FD, FD), Offset.reshape(B, 3, FD * FD, FD))

    return pl.pallas_call(
        _nms_kernel,
        out_shape=jax.ShapeDtypeStruct((B, TOPK, 8), jnp.float32),
    )(ts, boxes)


# speedup vs baseline: 2.5666x; 1.0003x over previous
import jax
import jax.numpy as jnp
from jax.experimental import pallas as pl
from jax.experimental.pallas import tpu as pltpu

TOPK = 60
THRESHOLD = 0.15
NMS_TH = 0.05
NMS_TOPK = 20
REG = 36
NCH = 3 * REG
FD = 24
N = FD * FD * FD
STRIDE = 4.0
NEG = -jnp.inf
BIG = 1 << 30
PAD = 64


def _topk_kernel(cls_ref, ts_ref, idx_ref):
    B = cls_ref.shape[0]
    S = jax.nn.sigmoid(cls_ref[...])
    r3 = jax.lax.broadcasted_iota(jnp.int32, (B, NCH, 128), 1)
    l3 = jax.lax.broadcasted_iota(jnp.int32, (B, NCH, 128), 2)
    n3 = r3 * 128 + l3
    lane = jax.lax.broadcasted_iota(jnp.int32, (B, PAD), 1)

    def body(k, carry):
        S, tsv, idxv = carry
        m = jnp.max(jnp.max(S, axis=1), axis=1, keepdims=True)
        cand = jnp.where(S == m[:, None, :], n3, BIG)
        i = jnp.min(jnp.min(cand, axis=1), axis=1, keepdims=True)
        S = jnp.where(n3 == i[:, None, :], NEG, S)
        tsv = jnp.where(lane == k, m, tsv)
        idxv = jnp.where(lane == k, i, idxv)
        return S, tsv, idxv

    tsv0 = jnp.full((B, PAD), NEG, jnp.float32)
    idxv0 = jnp.zeros((B, PAD), jnp.int32)
    _, tsv, idxv = jax.lax.fori_loop(0, TOPK, body, (S, tsv0, idxv0))
    ts_ref[...] = tsv
    idx_ref[...] = idxv


def _gather_kernel(idx_sref, idxv_ref, shape_hbm, off_hbm, boxes_ref,
                   gsh, gof, sem):
    jj = pl.program_id(0)
    copies = []
    for k in range(TOPK):
        ck = idx_sref[jj, k] // FD
        cp = pltpu.make_async_copy(
            shape_hbm.at[jj, :, ck, :], gsh.at[:, k, :], sem)
        cp.start()
        copies.append(cp)
        cpo = pltpu.make_async_copy(
            off_hbm.at[jj, :, ck, :], gof.at[:, k, :], sem)
        cpo.start()
        copies.append(cpo)
    gsh[:, TOPK:, :] = jnp.zeros((NCH, PAD - TOPK, FD), jnp.float32)
    gof[:, TOPK:, :] = jnp.zeros((3, PAD - TOPK, FD), jnp.float32)
    for cp in copies:
        cp.wait()

    idxs = idxv_ref[0]
    xsel = (jax.lax.broadcasted_iota(jnp.int32, (1, PAD, FD), 2)
            == (idxs % FD)[:, :, None]).astype(jnp.float32)
    dsel = jnp.sum(gsh[...] * xsel, axis=2)
    off_sel = jnp.sum(gof[...] * xsel, axis=2)

    proj_iota = jax.lax.broadcasted_iota(
        jnp.int32, (REG, PAD), 0).astype(jnp.float32)

    def project_rows(c):
        blk = dsel[c * REG:(c + 1) * REG, :]
        m = jnp.max(blk, axis=0, keepdims=True)
        e = jnp.exp(blk - m)
        p = e / jnp.sum(e, axis=0, keepdims=True)
        return jnp.sum(p * proj_iota, axis=0, keepdims=True)

    az = (idxs // (FD * FD)).astype(jnp.float32)
    ay = ((idxs // FD) % FD).astype(jnp.float32)
    ax = (idxs % FD).astype(jnp.float32)

    cz = (az + off_sel[0:1, :]) * STRIDE
    cy = (ay + off_sel[1:2, :]) * STRIDE
    cx = (ax + off_sel[2:3, :]) * STRIDE
    sz = project_rows(0) * STRIDE
    sy = project_rows(1) * STRIDE
    sx = project_rows(2) * STRIDE

    boxes_ref[0, 0:1, :] = cz
    boxes_ref[0, 1:2, :] = cy
    boxes_ref[0, 2:3, :] = cx
    boxes_ref[0, 3:4, :] = sz
    boxes_ref[0, 4:5, :] = sy
    boxes_ref[0, 5:6, :] = sx
    boxes_ref[0, 6:8, :] = jnp.zeros((2, PAD), jnp.float32)


def _nms_kernel(ts_ref, boxes_ref, out_ref):
    B = ts_ref.shape[0]
    ts = ts_ref[...]
    cz = boxes_ref[:, 0, :]
    cy = boxes_ref[:, 1, :]
    cx = boxes_ref[:, 2, :]
    sz = boxes_ref[:, 3, :]
    sy = boxes_ref[:, 4, :]
    sx = boxes_ref[:, 5, :]
    loz, hiz = cz - sz * 0.5, cz + sz * 0.5
    loy, hiy = cy - sy * 0.5, cy + sy * 0.5
    lox, hix = cx - sx * 0.5, cx + sx * 0.5
    vol = sz * sy * sx

    lane = jax.lax.broadcasted_iota(jnp.int32, (B, PAD), 1)
    col8 = jax.lax.broadcasted_iota(jnp.int32, (1, 1, 8), 2)
    out_ref[...] = jnp.full((B, TOPK, 8), -1.0, jnp.float32)

    s_cur = jnp.where(ts > THRESHOLD, ts, NEG)
    for k in range(NMS_TOPK):
        m = jnp.max(s_cur, axis=1, keepdims=True)
        i = jnp.min(jnp.where(s_cur == m, lane, BIG), axis=1, keepdims=True)
        ok = m > NEG
        sel = lane == i

        def pick(v):
            return jnp.sum(jnp.where(sel, v, 0.0), axis=1, keepdims=True)

        bz, by, bx = pick(cz), pick(cy), pick(cx)
        bsz, bsy, bsx = pick(sz), pick(sy), pick(sx)
        iz = jnp.clip(jnp.minimum(bz + bsz * 0.5, hiz)
                      - jnp.maximum(bz - bsz * 0.5, loz), 0.0, None)
        iy = jnp.clip(jnp.minimum(by + bsy * 0.5, hiy)
                      - jnp.maximum(by - bsy * 0.5, loy), 0.0, None)
        ix = jnp.clip(jnp.minimum(bx + bsx * 0.5, hix)
                      - jnp.maximum(bx - bsx * 0.5, lox), 0.0, None)
        inter = iz * iy * ix
        iou = inter / (bsz * bsy * bsx + vol - inter + 1e-8)
        supp = (iou > NMS_TH) | sel
        s_cur = jnp.where(ok & supp, NEG, s_cur)

        def col(c, v):
            return jnp.where(col8 == c, v[:, :, None], 0.0)

        vals = (col(0, jnp.ones_like(m)) + col(1, m) + col(2, bz)
                + col(3, by) + col(4, bx) + col(5, bsz) + col(6, bsy)
                + col(7, bsx))
        out_ref[:, k:k + 1, :] = jnp.where(ok[:, :, None], vals, -1.0)


@jax.jit
def kernel(Cls, Shape, Offset):
    B = Cls.shape[0]
    cls_r = Cls.reshape(B, NCH, 128)

    ts, idx = pl.pallas_call(
        _topk_kernel,
        out_shape=(jax.ShapeDtypeStruct((B, PAD), jnp.float32),
                   jax.ShapeDtypeStruct((B, PAD), jnp.int32)),
    )(cls_r)

    boxes = pl.pallas_call(
        _gather_kernel,
        grid_spec=pltpu.PrefetchScalarGridSpec(
            num_scalar_prefetch=1,
            grid=(B,),
            in_specs=[
                pl.BlockSpec((1, 1, PAD), lambda j, iref: (j, 0, 0)),
                pl.BlockSpec(memory_space=pltpu.MemorySpace.HBM),
                pl.BlockSpec(memory_space=pltpu.MemorySpace.HBM),
            ],
            out_specs=pl.BlockSpec((1, 8, PAD), lambda j, iref: (j, 0, 0)),
            scratch_shapes=[
                pltpu.VMEM((NCH, PAD, FD), jnp.float32),
                pltpu.VMEM((3, PAD, FD), jnp.float32),
                pltpu.SemaphoreType.DMA,
            ],
        ),
        out_shape=jax.ShapeDtypeStruct((B, 8, PAD), jnp.float32),
        compiler_params=pltpu.CompilerParams(
            dimension_semantics=("parallel",)),
    )(idx, idx.reshape(B, 1, PAD),
      Shape.reshape(B, NCH, FD * FD, FD), Offset.reshape(B, 3, FD * FD, FD))

    return pl.pallas_call(
        _nms_kernel,
        out_shape=jax.ShapeDtypeStruct((B, TOPK, 8), jnp.float32),
    )(ts, boxes)
